# baseline (speedup 1.0000x reference)
import functools

import jax
import jax.numpy as jnp
from jax import lax
from jax.experimental import pallas as pl
from jax.experimental.pallas import tpu as pltpu
from jax.experimental.pallas import tpu_sc as plsc

_NC = 2
_NS = 16
_NW = _NC * _NS

_N = 10000
_D = 128
_E = 320000

_CHUNK = 50
_EPT = _E // _NW
_NCHUNK = _EPT // _CHUNK
_NBUF = 5
_RCHUNK = 80
_NRCHUNK = _N // _RCHUNK
_CNTW = 128
_IBLK = 40
_NBLK = _NCHUNK // _IBLK
_ZR = 40
_NZCHUNK = _N // _ZR

_f32 = jnp.float32


def _make_layer_body(with_counts, nbuf):
  def body(y_hbm, src_hbm, dst_hbm, *rest):
    if with_counts:
      s_out, cnt_out = rest[0], rest[1]
      scr = rest[2:]
    else:
      s_out = rest[0]
      scr = rest[1:]
    cid = lax.axis_index("c")
    sid = lax.axis_index("s")
    wid = cid * _NS + sid
    src_v, dst_v = scr[0], scr[1]
    rows = list(scr[2:2 + nbuf])
    acc_sh = scr[2 + nbuf]
    pos = 3 + nbuf
    if with_counts:
      ones_v = scr[pos]
      pos += 1
    gsem = list(scr[pos:pos + nbuf])
    ssem = list(scr[pos + nbuf:pos + 2 * nbuf])

    def _fill_zero_rows():
      def _fill(i, _):
        for j in range(_D // 16):
          rows[0][i, pl.ds(j * 16, 16)] = jnp.zeros((16,), _f32)
        return 0
      lax.fori_loop(0, _ZR, _fill, 0)

    def _zero_table():
      zsrc = rows[0].at[pl.ds(0, _ZR)]

      def _zero_chunk(i, _):
        c = sid + _NS * i

        @pl.when(c < _NZCHUNK)
        def _():
          pltpu.sync_copy(zsrc, acc_sh.at[pl.ds(c * _ZR, _ZR)])
        return 0
      lax.fori_loop(0, pl.cdiv(_NZCHUNK, _NS), _zero_chunk, 0)

    def _drain_table(dst_hbm_slice):
      def _drain_chunk(i, _):
        c = sid + _NS * i

        @pl.when(c < _NRCHUNK)
        def _():
          rows_sl = pl.ds(c * _RCHUNK, _RCHUNK)
          pltpu.sync_copy(acc_sh.at[rows_sl], dst_hbm_slice.at[rows_sl])
        return 0
      lax.fori_loop(0, pl.cdiv(_NRCHUNK, _NS), _drain_chunk, 0)

    _fill_zero_rows()
    if with_counts:
      def _fill_o(i, _):
        for j in range(_CNTW // 16):
          ones_v[i, pl.ds(j * 16, 16)] = jnp.ones((16,), _f32)
        return 0
      lax.fori_loop(0, _CHUNK, _fill_o, 0)

      for r in range(4):
        _zero_table()
        plsc.subcore_barrier()

        def _idx_block(b, _):
          pltpu.sync_copy(dst_hbm.at[r, wid, b], dst_v)
          for k in range(nbuf):
            pltpu.async_copy(ones_v, acc_sh.at[dst_v.at[k]], ssem[k],
                             add=True)

          def _group(i, _):
            base = i * nbuf
            for k in range(nbuf):
              c = base + k + nbuf

              @pl.when(c < _IBLK)
              def _():
                pltpu.make_async_copy(
                    ones_v, acc_sh.at[dst_v.at[c]], ssem[k]).wait()
                pltpu.async_copy(ones_v, acc_sh.at[dst_v.at[c]], ssem[k],
                                 add=True)
            return 0
          lax.fori_loop(0, _IBLK // nbuf, _group, 0)

          for k in range(nbuf):
            pltpu.make_async_copy(
                ones_v, acc_sh.at[dst_v.at[0]], ssem[k]).wait()
          return 0
        lax.fori_loop(0, _NBLK, _idx_block, 0)
        plsc.subcore_barrier()
        _drain_table(cnt_out.at[r, cid])
        plsc.subcore_barrier()

    for r in range(4):
      _fill_zero_rows()
      _zero_table()
      plsc.subcore_barrier()

      def _idx_block(b, _):
        pltpu.sync_copy(src_hbm.at[r, wid, b], src_v)
        pltpu.sync_copy(dst_hbm.at[r, wid, b], dst_v)
        for k in range(nbuf):
          pltpu.async_copy(y_hbm.at[r].at[src_v.at[k]], rows[k], gsem[k])

        def _group(i, _):
          base = i * nbuf
          for k in range(nbuf):
            c = base + k
            pltpu.make_async_copy(
                y_hbm.at[r].at[src_v.at[c]], rows[k], gsem[k]).wait()
            pltpu.async_copy(rows[k], acc_sh.at[dst_v.at[c]], ssem[k],
                             add=True)
          for k in range(nbuf):
            c = base + k + nbuf

            @pl.when(c < _IBLK)
            def _():
              pltpu.make_async_copy(
                  rows[k], acc_sh.at[dst_v.at[c]], ssem[k]).wait()
              pltpu.async_copy(y_hbm.at[r].at[src_v.at[c]], rows[k], gsem[k])
          return 0
        lax.fori_loop(0, _IBLK // nbuf, _group, 0)

        for k in range(nbuf):
          pltpu.make_async_copy(
              rows[k], acc_sh.at[dst_v.at[0]], ssem[k]).wait()
        return 0
      lax.fori_loop(0, _NBLK, _idx_block, 0)
      plsc.subcore_barrier()
      _drain_table(s_out.at[r, cid])
      plsc.subcore_barrier()

  return body


def _make_layer_kernel(with_counts, nbuf):
  out_type = jax.ShapeDtypeStruct((4, _NC, _N, _D), _f32)
  if with_counts:
    out_type = [out_type, jax.ShapeDtypeStruct((4, _NC, _N, _CNTW), _f32)]
  scratch = (
      [pltpu.VMEM((_IBLK, _CHUNK), jnp.int32)] * 2
      + [pltpu.VMEM((_CHUNK, _D), _f32)] * nbuf
      + [pltpu.VMEM_SHARED((_N, _D), _f32)]
      + ([pltpu.VMEM((_CHUNK, _CNTW), _f32)] if with_counts else [])
      + [pltpu.SemaphoreType.DMA] * (2 * nbuf)
  )
  return pl.kernel(
      _make_layer_body(with_counts, nbuf),
      out_type=out_type,
      mesh=plsc.VectorSubcoreMesh(core_axis_name="c", subcore_axis_name="s"),
      scratch_types=scratch,
  )


_layer0 = _make_layer_kernel(True, _NBUF - 1)
_layer1 = _make_layer_kernel(False, _NBUF)


def _mm4_body(x_ref, w_ref, o_ref):
  o_ref[0] = jnp.dot(x_ref[0], w_ref[0], preferred_element_type=_f32)


_mm4 = pl.pallas_call(
    _mm4_body,
    grid=(4,),
    in_specs=[
        pl.BlockSpec((1, _N, _D), lambda i: (i, 0, 0)),
        pl.BlockSpec((1, _D, _D), lambda i: (i, 0, 0)),
    ],
    out_specs=pl.BlockSpec((1, _N, _D), lambda i: (i, 0, 0)),
    out_shape=jax.ShapeDtypeStruct((4, _N, _D), _f32),
)


def _leaky(v):
  return jnp.where(v >= 0, v, 0.01 * v)


def _comb1_body(s_ref, c_ref, x_ref, w_ref, b_ref, o_ref):
  stot = s_ref[0] + s_ref[1]
  inv = 1.0 / jnp.maximum(c_ref[0] + c_ref[1], 1.0)
  v = (stot * inv + b_ref[...]
       + jnp.dot(x_ref[...], w_ref[...], preferred_element_type=_f32))
  o_ref[...] = _leaky(v)


_comb1 = pl.pallas_call(
    _comb1_body,
    out_shape=jax.ShapeDtypeStruct((_N, _D), _f32),
)


def _comb2_body(sa_ref, ca_ref, sb_ref, cb_ref, x_ref, w_ref, b_ref, o_ref):
  sa = sa_ref[0] + sa_ref[1]
  ca = ca_ref[0] + ca_ref[1]
  sb = sb_ref[0] + sb_ref[1]
  cb = cb_ref[0] + cb_ref[1]
  v = (sa / jnp.maximum(ca, 1.0) + sb / jnp.maximum(cb, 1.0) + b_ref[...]
       + jnp.dot(x_ref[...], w_ref[...], preferred_element_type=_f32))
  o_ref[...] = _leaky(v)


_comb2 = pl.pallas_call(
    _comb2_body,
    out_shape=jax.ShapeDtypeStruct((_N, _D), _f32),
)


def kernel(x_node1, x_node2, x_node3, edge_index_node2_to_node3,
           edge_index_node1_to_node2, edge_index_node3_rev_to_node2,
           edge_index_node2_rev_to_node1, params):
  edges = [edge_index_node2_to_node3, edge_index_node1_to_node2,
           edge_index_node3_rev_to_node2, edge_index_node2_rev_to_node1]
  srcs, dsts = [], []
  for e in edges:
    e = e.astype(jnp.int32)
    srcs.append(e[0].reshape(_NW, _NBLK, _IBLK, _CHUNK))
    dsts.append(e[1].reshape(_NW, _NBLK, _IBLK, _CHUNK))
  src4 = jnp.stack(srcs)
  dst4 = jnp.stack(dsts)

  rel = [(1, 2), (0, 1), (2, 1), (1, 0)]
  xs = (x_node1, x_node2, x_node3)
  cnt4 = None

  for li in range(2):
    lp = params["layer%d" % li]
    x_stack = jnp.stack([xs[rel[r][0]] for r in range(4)])
    w_stack = jnp.stack([lp["e%d" % r]["W_l"] for r in range(4)])
    y4 = _mm4(x_stack, w_stack)
    if li == 0:
      s4, cnt4 = _layer0(y4, src4, dst4)
    else:
      s4 = _layer1(y4, src4, dst4)

    b = [lp["e%d" % r]["b_l"].reshape(1, _D) for r in range(4)]
    new1 = _comb1(s4[3], cnt4[3], xs[0], lp["e3"]["W_r"], b[3])
    new2 = _comb2(s4[1], cnt4[1], s4[2], cnt4[2], xs[1],
                  lp["e1"]["W_r"] + lp["e2"]["W_r"], b[1] + b[2])
    new3 = _comb1(s4[0], cnt4[0], xs[2], lp["e0"]["W_r"], b[0])
    xs = (new1, new2, new3)

  return xs


# speedup vs baseline: 1.1572x; 1.1572x over previous
import jax
import jax.numpy as jnp
from jax import lax
from jax.experimental import pallas as pl
from jax.experimental.pallas import tpu as pltpu
from jax.experimental.pallas import tpu_sc as plsc

_NC = 2
_NS = 16
_NW = _NC * _NS

_N = 10000
_D = 128
_E = 320000

_CHUNK = 50
_EPT = _E // _NW
_NCHUNK = _EPT // _CHUNK
_NBUF = 5
_RCHUNK = 80
_NRCHUNK = _N // _RCHUNK
_CNTW = 128
_IBLK = 40
_NBLK = _NCHUNK // _IBLK
_ZR = 40
_NZCHUNK = _N // _ZR

_f32 = jnp.float32


def _zero_table(sid, zsrc, table_sh, sems):
  nsem = len(sems)
  niter = pl.cdiv(_NZCHUNK, _NS)
  for i in range(niter):
    if i >= nsem:
      cprev = sid + _NS * (i - nsem)

      @pl.when(cprev < _NZCHUNK)
      def _():
        pltpu.make_async_copy(zsrc, table_sh.at[pl.ds(0, _ZR)],
                              sems[i % nsem]).wait()
    c = sid + _NS * i

    @pl.when(c < _NZCHUNK)
    def _():
      pltpu.async_copy(zsrc, table_sh.at[pl.ds(c * _ZR, _ZR)], sems[i % nsem])
  for i in range(max(0, niter - nsem), niter):
    c = sid + _NS * i

    @pl.when(c < _NZCHUNK)
    def _():
      pltpu.make_async_copy(zsrc, table_sh.at[pl.ds(0, _ZR)],
                            sems[i % nsem]).wait()


def _drain_table(sid, table_sh, out_slice, sems):
  nsem = len(sems)
  niter = pl.cdiv(_NRCHUNK, _NS)
  dummy = pl.ds(0, _RCHUNK)
  for i in range(niter):
    if i >= nsem:
      cprev = sid + _NS * (i - nsem)

      @pl.when(cprev < _NRCHUNK)
      def _():
        pltpu.make_async_copy(table_sh.at[dummy], out_slice.at[dummy],
                              sems[i % nsem]).wait()
    c = sid + _NS * i

    @pl.when(c < _NRCHUNK)
    def _():
      rows_sl = pl.ds(c * _RCHUNK, _RCHUNK)
      pltpu.async_copy(table_sh.at[rows_sl], out_slice.at[rows_sl],
                       sems[i % nsem])
  for i in range(max(0, niter - nsem), niter):
    c = sid + _NS * i

    @pl.when(c < _NRCHUNK)
    def _():
      pltpu.make_async_copy(table_sh.at[dummy], out_slice.at[dummy],
                            sems[i % nsem]).wait()


def _segsum_body(y_hbm, src_hbm, dst_hbm, s_out, *scr):
  cid = lax.axis_index("c")
  sid = lax.axis_index("s")
  wid = cid * _NS + sid
  src_v, dst_v = scr[0], scr[1]
  rows = list(scr[2:2 + _NBUF])
  acc_sh = scr[2 + _NBUF]
  gsem = list(scr[3 + _NBUF:3 + 2 * _NBUF])
  ssem = list(scr[3 + 2 * _NBUF:3 + 3 * _NBUF])

  def _fill_row(i, _):
    for j in range(_D // 16):
      rows[0][i, pl.ds(j * 16, 16)] = jnp.zeros((16,), _f32)
    return 0
  lax.fori_loop(0, _ZR, _fill_row, 0)

  _zero_table(sid, rows[0].at[pl.ds(0, _ZR)], acc_sh, ssem)
  plsc.subcore_barrier()

  def _idx_block(b, _):
    pltpu.async_copy(src_hbm.at[wid, b], src_v, gsem[0])
    pltpu.async_copy(dst_hbm.at[wid, b], dst_v, gsem[1])
    pltpu.make_async_copy(src_hbm.at[wid, b], src_v, gsem[0]).wait()
    pltpu.make_async_copy(dst_hbm.at[wid, b], dst_v, gsem[1]).wait()
    for k in range(_NBUF):
      pltpu.async_copy(y_hbm.at[src_v.at[k]], rows[k], gsem[k])

    def _group(i, _):
      base = i * _NBUF
      for k in range(_NBUF):
        c = base + k
        pltpu.make_async_copy(y_hbm.at[src_v.at[c]], rows[k], gsem[k]).wait()
        pltpu.async_copy(rows[k], acc_sh.at[dst_v.at[c]], ssem[k], add=True)
      for k in range(_NBUF):
        c = base + k + _NBUF

        @pl.when(c < _IBLK)
        def _():
          pltpu.make_async_copy(
              rows[k], acc_sh.at[dst_v.at[c]], ssem[k]).wait()
          pltpu.async_copy(y_hbm.at[src_v.at[c]], rows[k], gsem[k])
      return 0
    lax.fori_loop(0, _IBLK // _NBUF, _group, 0)

    for k in range(_NBUF):
      pltpu.make_async_copy(
          rows[k], acc_sh.at[dst_v.at[0]], ssem[k]).wait()
    return 0
  lax.fori_loop(0, _NBLK, _idx_block, 0)
  plsc.subcore_barrier()

  _drain_table(sid, acc_sh, s_out.at[cid], ssem)


_segsum = pl.kernel(
    _segsum_body,
    out_type=jax.ShapeDtypeStruct((_NC, _N, _D), _f32),
    mesh=plsc.VectorSubcoreMesh(core_axis_name="c", subcore_axis_name="s"),
    scratch_types=(
        [pltpu.VMEM((_IBLK, _CHUNK), jnp.int32)] * 2
        + [pltpu.VMEM((_CHUNK, _D), _f32)] * _NBUF
        + [pltpu.VMEM_SHARED((_N, _D), _f32)]
        + [pltpu.SemaphoreType.DMA] * (2 * _NBUF)
    ),
)


def _count_body(dst_hbm, cnt_out, *scr):
  cid = lax.axis_index("c")
  sid = lax.axis_index("s")
  wid = cid * _NS + sid
  dst_v, ones_v, zcnt_v = scr[0], scr[1], scr[2]
  cnt_sh = scr[3]
  ssem = list(scr[4:4 + _NBUF])

  def _fill_z(i, _):
    for j in range(_CNTW // 16):
      zcnt_v[i, pl.ds(j * 16, 16)] = jnp.zeros((16,), _f32)
    return 0
  lax.fori_loop(0, _ZR, _fill_z, 0)

  def _fill_o(i, _):
    for j in range(_CNTW // 16):
      ones_v[i, pl.ds(j * 16, 16)] = jnp.ones((16,), _f32)
    return 0
  lax.fori_loop(0, _CHUNK, _fill_o, 0)

  for r in range(4):
    _zero_table(sid, zcnt_v.at[pl.ds(0, _ZR)], cnt_sh, ssem)
    plsc.subcore_barrier()

    def _idx_block(b, _):
      pltpu.sync_copy(dst_hbm.at[r, wid, b], dst_v)
      for k in range(_NBUF):
        pltpu.async_copy(ones_v, cnt_sh.at[dst_v.at[k]], ssem[k], add=True)

      def _group(i, _):
        base = i * _NBUF
        for k in range(_NBUF):
          c = base + k + _NBUF

          @pl.when(c < _IBLK)
          def _():
            pltpu.make_async_copy(
                ones_v, cnt_sh.at[dst_v.at[c]], ssem[k]).wait()
            pltpu.async_copy(ones_v, cnt_sh.at[dst_v.at[c]], ssem[k], add=True)
        return 0
      lax.fori_loop(0, _IBLK // _NBUF, _group, 0)

      for k in range(_NBUF):
        pltpu.make_async_copy(ones_v, cnt_sh.at[dst_v.at[0]], ssem[k]).wait()
      return 0
    lax.fori_loop(0, _NBLK, _idx_block, 0)
    plsc.subcore_barrier()

    _drain_table(sid, cnt_sh, cnt_out.at[r, cid], ssem)
    plsc.subcore_barrier()


_count = pl.kernel(
    _count_body,
    out_type=jax.ShapeDtypeStruct((4, _NC, _N, _CNTW), _f32),
    mesh=plsc.VectorSubcoreMesh(core_axis_name="c", subcore_axis_name="s"),
    scratch_types=(
        [pltpu.VMEM((_IBLK, _CHUNK), jnp.int32)]
        + [pltpu.VMEM((_CHUNK, _CNTW), _f32)]
        + [pltpu.VMEM((_ZR, _CNTW), _f32)]
        + [pltpu.VMEM_SHARED((_N, _CNTW), _f32)]
        + [pltpu.SemaphoreType.DMA] * _NBUF
    ),
)


def _mm_body(x_ref, w_ref, o_ref):
  o_ref[...] = jnp.dot(x_ref[...], w_ref[...], preferred_element_type=_f32)


_mm = pl.pallas_call(
    _mm_body,
    out_shape=jax.ShapeDtypeStruct((_N, _D), _f32),
)


def _leaky(v):
  return jnp.where(v >= 0, v, 0.01 * v)


def _comb1_body(s_ref, c_ref, x_ref, w_ref, b_ref, o_ref):
  stot = s_ref[0] + s_ref[1]
  inv = 1.0 / jnp.maximum(c_ref[0] + c_ref[1], 1.0)
  v = (stot * inv + b_ref[...]
       + jnp.dot(x_ref[...], w_ref[...], preferred_element_type=_f32))
  o_ref[...] = _leaky(v)


_comb1 = pl.pallas_call(
    _comb1_body,
    out_shape=jax.ShapeDtypeStruct((_N, _D), _f32),
)


def _comb2_body(sa_ref, ca_ref, sb_ref, cb_ref, x_ref, w_ref, b_ref, o_ref):
  sa = sa_ref[0] + sa_ref[1]
  ca = ca_ref[0] + ca_ref[1]
  sb = sb_ref[0] + sb_ref[1]
  cb = cb_ref[0] + cb_ref[1]
  v = (sa / jnp.maximum(ca, 1.0) + sb / jnp.maximum(cb, 1.0) + b_ref[...]
       + jnp.dot(x_ref[...], w_ref[...], preferred_element_type=_f32))
  o_ref[...] = _leaky(v)


_comb2 = pl.pallas_call(
    _comb2_body,
    out_shape=jax.ShapeDtypeStruct((_N, _D), _f32),
)


def kernel(x_node1, x_node2, x_node3, edge_index_node2_to_node3,
           edge_index_node1_to_node2, edge_index_node3_rev_to_node2,
           edge_index_node2_rev_to_node1, params):
  edges = [edge_index_node2_to_node3, edge_index_node1_to_node2,
           edge_index_node3_rev_to_node2, edge_index_node2_rev_to_node1]
  srcs, dsts = [], []
  for e in edges:
    e = e.astype(jnp.int32)
    srcs.append(e[0].reshape(_NW, _NBLK, _IBLK, _CHUNK))
    dsts.append(e[1].reshape(_NW, _NBLK, _IBLK, _CHUNK))

  cnt = _count(jnp.stack(dsts))

  rel = [(1, 2), (0, 1), (2, 1), (1, 0)]
  xs = (x_node1, x_node2, x_node3)

  for li in range(2):
    lp = params["layer%d" % li]
    segs = []
    for r in range(4):
      p = lp["e%d" % r]
      y = _mm(xs[rel[r][0]], p["W_l"])
      segs.append(_segsum(y, srcs[r], dsts[r]))

    b = [lp["e%d" % r]["b_l"].reshape(1, _D) for r in range(4)]
    new1 = _comb1(segs[3], cnt[3], xs[0], lp["e3"]["W_r"], b[3])
    new2 = _comb2(segs[1], cnt[1], segs[2], cnt[2], xs[1],
                  lp["e1"]["W_r"] + lp["e2"]["W_r"], b[1] + b[2])
    new3 = _comb1(segs[0], cnt[0], xs[2], lp["e0"]["W_r"], b[0])
    xs = (new1, new2, new3)

  return xs


# speedup vs baseline: 1.1747x; 1.0152x over previous
import jax
import jax.numpy as jnp
from jax import lax
from jax.experimental import pallas as pl
from jax.experimental.pallas import tpu as pltpu
from jax.experimental.pallas import tpu_sc as plsc

_NC = 2
_NS = 16
_NW = _NC * _NS

_N = 10000
_D = 128
_E = 320000

_CHUNK = 50
_EPT = _E // _NW
_NCHUNK = _EPT // _CHUNK
_NBUF = 5
_RCHUNK = 80
_NRCHUNK = _N // _RCHUNK
_CNTW = 128
_IBLK = 40
_NBLK = _NCHUNK // _IBLK
_ZR = 40
_NZCHUNK = _N // _ZR

_CCHUNK = 125
_CIBLK = 16
_CNBLK = (_EPT // _CCHUNK) // _CIBLK

_f32 = jnp.float32


def _zero_table(sid, zsrc, table_sh, sems):
  nsem = len(sems)
  niter = pl.cdiv(_NZCHUNK, _NS)
  for i in range(niter):
    if i >= nsem:
      cprev = sid + _NS * (i - nsem)

      @pl.when(cprev < _NZCHUNK)
      def _():
        pltpu.make_async_copy(zsrc, table_sh.at[pl.ds(0, _ZR)],
                              sems[i % nsem]).wait()
    c = sid + _NS * i

    @pl.when(c < _NZCHUNK)
    def _():
      pltpu.async_copy(zsrc, table_sh.at[pl.ds(c * _ZR, _ZR)], sems[i % nsem])
  for i in range(max(0, niter - nsem), niter):
    c = sid + _NS * i

    @pl.when(c < _NZCHUNK)
    def _():
      pltpu.make_async_copy(zsrc, table_sh.at[pl.ds(0, _ZR)],
                            sems[i % nsem]).wait()


def _drain_table(sid, table_sh, out_slice, sems):
  nsem = len(sems)
  niter = pl.cdiv(_NRCHUNK, _NS)
  dummy = pl.ds(0, _RCHUNK)
  for i in range(niter):
    if i >= nsem:
      cprev = sid + _NS * (i - nsem)

      @pl.when(cprev < _NRCHUNK)
      def _():
        pltpu.make_async_copy(table_sh.at[dummy], out_slice.at[dummy],
                              sems[i % nsem]).wait()
    c = sid + _NS * i

    @pl.when(c < _NRCHUNK)
    def _():
      rows_sl = pl.ds(c * _RCHUNK, _RCHUNK)
      pltpu.async_copy(table_sh.at[rows_sl], out_slice.at[rows_sl],
                       sems[i % nsem])
  for i in range(max(0, niter - nsem), niter):
    c = sid + _NS * i

    @pl.when(c < _NRCHUNK)
    def _():
      pltpu.make_async_copy(table_sh.at[dummy], out_slice.at[dummy],
                            sems[i % nsem]).wait()


def _segsum_body(y_hbm, src_hbm, dst_hbm, s_out, *scr):
  cid = lax.axis_index("c")
  sid = lax.axis_index("s")
  wid = cid * _NS + sid
  src_v, dst_v = scr[0], scr[1]
  rows = list(scr[2:2 + _NBUF])
  acc_sh = scr[2 + _NBUF]
  gsem = list(scr[3 + _NBUF:3 + 2 * _NBUF])
  ssem = list(scr[3 + 2 * _NBUF:3 + 3 * _NBUF])

  def _fill_row(i, _):
    for j in range(_D // 16):
      rows[0][i, pl.ds(j * 16, 16)] = jnp.zeros((16,), _f32)
    return 0
  lax.fori_loop(0, _ZR, _fill_row, 0)

  _zero_table(sid, rows[0].at[pl.ds(0, _ZR)], acc_sh, ssem)
  plsc.subcore_barrier()

  def _idx_block(b, _):
    pltpu.async_copy(src_hbm.at[wid, b], src_v, gsem[0])
    pltpu.async_copy(dst_hbm.at[wid, b], dst_v, gsem[1])
    pltpu.make_async_copy(src_hbm.at[wid, b], src_v, gsem[0]).wait()
    pltpu.make_async_copy(dst_hbm.at[wid, b], dst_v, gsem[1]).wait()
    for k in range(_NBUF):
      pltpu.async_copy(y_hbm.at[src_v.at[k]], rows[k], gsem[k])

    def _group(i, _):
      base = i * _NBUF
      for k in range(_NBUF):
        c = base + k
        pltpu.make_async_copy(y_hbm.at[src_v.at[c]], rows[k], gsem[k]).wait()
        pltpu.async_copy(rows[k], acc_sh.at[dst_v.at[c]], ssem[k], add=True)
      for k in range(_NBUF):
        c = base + k + _NBUF

        @pl.when(c < _IBLK)
        def _():
          pltpu.make_async_copy(
              rows[k], acc_sh.at[dst_v.at[c]], ssem[k]).wait()
          pltpu.async_copy(y_hbm.at[src_v.at[c]], rows[k], gsem[k])
      return 0
    lax.fori_loop(0, _IBLK // _NBUF, _group, 0)

    for k in range(_NBUF):
      pltpu.make_async_copy(
          rows[k], acc_sh.at[dst_v.at[0]], ssem[k]).wait()
    return 0
  lax.fori_loop(0, _NBLK, _idx_block, 0)
  plsc.subcore_barrier()

  _drain_table(sid, acc_sh, s_out.at[cid], ssem)


_segsum = pl.kernel(
    _segsum_body,
    out_type=jax.ShapeDtypeStruct((_NC, _N, _D), _f32),
    mesh=plsc.VectorSubcoreMesh(core_axis_name="c", subcore_axis_name="s"),
    scratch_types=(
        [pltpu.VMEM((_IBLK, _CHUNK), jnp.int32)] * 2
        + [pltpu.VMEM((_CHUNK, _D), _f32)] * _NBUF
        + [pltpu.VMEM_SHARED((_N, _D), _f32)]
        + [pltpu.SemaphoreType.DMA] * (2 * _NBUF)
    ),
)


def _count_body(dst_hbm, cnt_out, *scr):
  cid = lax.axis_index("c")
  sid = lax.axis_index("s")
  wid = cid * _NS + sid
  dst_v, ones_v, zcnt_v = scr[0], scr[1], scr[2]
  cnt_sh = scr[3]
  ssem = list(scr[4:4 + _NBUF])

  def _fill_z(i, _):
    for j in range(_CNTW // 16):
      zcnt_v[i, pl.ds(j * 16, 16)] = jnp.zeros((16,), _f32)
    return 0
  lax.fori_loop(0, _ZR, _fill_z, 0)

  def _fill_o(i, _):
    for j in range(_CNTW // 16):
      ones_v[i, pl.ds(j * 16, 16)] = jnp.ones((16,), _f32)
    return 0
  lax.fori_loop(0, _CCHUNK, _fill_o, 0)

  for r in range(4):
    _zero_table(sid, zcnt_v.at[pl.ds(0, _ZR)], cnt_sh, ssem)
    plsc.subcore_barrier()

    def _idx_block(b, _):
      pltpu.sync_copy(dst_hbm.at[r, wid, b], dst_v)
      for k in range(_NBUF):
        pltpu.async_copy(ones_v, cnt_sh.at[dst_v.at[k]], ssem[k], add=True)

      def _group(i, _):
        base = i * _NBUF
        for k in range(_NBUF):
          c = base + k + _NBUF

          @pl.when(c < _CIBLK)
          def _():
            pltpu.make_async_copy(
                ones_v, cnt_sh.at[dst_v.at[c]], ssem[k]).wait()
            pltpu.async_copy(ones_v, cnt_sh.at[dst_v.at[c]], ssem[k], add=True)
        return 0
      lax.fori_loop(0, _CIBLK // _NBUF, _group, 0)

      for k in range(_NBUF):
        pltpu.make_async_copy(ones_v, cnt_sh.at[dst_v.at[0]], ssem[k]).wait()
      return 0
    lax.fori_loop(0, _CNBLK, _idx_block, 0)
    plsc.subcore_barrier()

    _drain_table(sid, cnt_sh, cnt_out.at[r, cid], ssem)
    plsc.subcore_barrier()


_count = pl.kernel(
    _count_body,
    out_type=jax.ShapeDtypeStruct((4, _NC, _N, _CNTW), _f32),
    mesh=plsc.VectorSubcoreMesh(core_axis_name="c", subcore_axis_name="s"),
    scratch_types=(
        [pltpu.VMEM((_CIBLK, _CCHUNK), jnp.int32)]
        + [pltpu.VMEM((_CCHUNK, _CNTW), _f32)]
        + [pltpu.VMEM((_ZR, _CNTW), _f32)]
        + [pltpu.VMEM_SHARED((_N, _CNTW), _f32)]
        + [pltpu.SemaphoreType.DMA] * _NBUF
    ),
)


def _mm_body(x_ref, w_ref, o_ref):
  o_ref[...] = jnp.dot(x_ref[...], w_ref[...], preferred_element_type=_f32)


_mm = pl.pallas_call(
    _mm_body,
    out_shape=jax.ShapeDtypeStruct((_N, _D), _f32),
)


def _leaky(v):
  return jnp.where(v >= 0, v, 0.01 * v)


def _comb1_body(s_ref, c_ref, x_ref, w_ref, b_ref, o_ref):
  stot = s_ref[0] + s_ref[1]
  inv = 1.0 / jnp.maximum(c_ref[0] + c_ref[1], 1.0)
  v = (stot * inv + b_ref[...]
       + jnp.dot(x_ref[...], w_ref[...], preferred_element_type=_f32))
  o_ref[...] = _leaky(v)


_comb1 = pl.pallas_call(
    _comb1_body,
    out_shape=jax.ShapeDtypeStruct((_N, _D), _f32),
)


def _comb2_body(sa_ref, ca_ref, sb_ref, cb_ref, x_ref, w_ref, b_ref, o_ref):
  sa = sa_ref[0] + sa_ref[1]
  ca = ca_ref[0] + ca_ref[1]
  sb = sb_ref[0] + sb_ref[1]
  cb = cb_ref[0] + cb_ref[1]
  v = (sa / jnp.maximum(ca, 1.0) + sb / jnp.maximum(cb, 1.0) + b_ref[...]
       + jnp.dot(x_ref[...], w_ref[...], preferred_element_type=_f32))
  o_ref[...] = _leaky(v)


_comb2 = pl.pallas_call(
    _comb2_body,
    out_shape=jax.ShapeDtypeStruct((_N, _D), _f32),
)


def kernel(x_node1, x_node2, x_node3, edge_index_node2_to_node3,
           edge_index_node1_to_node2, edge_index_node3_rev_to_node2,
           edge_index_node2_rev_to_node1, params):
  edges = [edge_index_node2_to_node3, edge_index_node1_to_node2,
           edge_index_node3_rev_to_node2, edge_index_node2_rev_to_node1]
  srcs, dsts, dsts_c = [], [], []
  for e in edges:
    e = e.astype(jnp.int32)
    srcs.append(e[0].reshape(_NW, _NBLK, _IBLK, _CHUNK))
    dsts.append(e[1].reshape(_NW, _NBLK, _IBLK, _CHUNK))
    dsts_c.append(e[1].reshape(_NW, _CNBLK, _CIBLK, _CCHUNK))

  cnt = _count(jnp.stack(dsts_c))

  rel = [(1, 2), (0, 1), (2, 1), (1, 0)]
  xs = (x_node1, x_node2, x_node3)

  for li in range(2):
    lp = params["layer%d" % li]
    segs = []
    for r in range(4):
      p = lp["e%d" % r]
      y = _mm(xs[rel[r][0]], p["W_l"])
      segs.append(_segsum(y, srcs[r], dsts[r]))

    b = [lp["e%d" % r]["b_l"].reshape(1, _D) for r in range(4)]
    new1 = _comb1(segs[3], cnt[3], xs[0], lp["e3"]["W_r"], b[3])
    new2 = _comb2(segs[1], cnt[1], segs[2], cnt[2], xs[1],
                  lp["e1"]["W_r"] + lp["e2"]["W_r"], b[1] + b[2])
    new3 = _comb1(segs[0], cnt[0], xs[2], lp["e0"]["W_r"], b[0])
    xs = (new1, new2, new3)

  return xs


# speedup vs baseline: 1.1837x; 1.0076x over previous
import jax
import jax.numpy as jnp
from jax import lax
from jax.experimental import pallas as pl
from jax.experimental.pallas import tpu as pltpu
from jax.experimental.pallas import tpu_sc as plsc

_NC = 2
_NS = 16
_NW = _NC * _NS

_N = 10000
_D = 128
_E = 320000

_CHUNK = 50
_EPT = _E // _NW
_NCHUNK = _EPT // _CHUNK
_NBUF = 5
_RCHUNK = 80
_NRCHUNK = _N // _RCHUNK
_CNTW = 128
_IBLK = 50
_NBLK = _NCHUNK // _IBLK
_ZR = 40
_NZCHUNK = _N // _ZR

_CCHUNK = 125
_CIBLK = 16
_CNBLK = (_EPT // _CCHUNK) // _CIBLK

_f32 = jnp.float32


def _zero_table(sid, zsrc, table_sh, sems):
  nsem = len(sems)
  niter = pl.cdiv(_NZCHUNK, _NS)
  for i in range(niter):
    if i >= nsem:
      cprev = sid + _NS * (i - nsem)

      @pl.when(cprev < _NZCHUNK)
      def _():
        pltpu.make_async_copy(zsrc, table_sh.at[pl.ds(0, _ZR)],
                              sems[i % nsem]).wait()
    c = sid + _NS * i

    @pl.when(c < _NZCHUNK)
    def _():
      pltpu.async_copy(zsrc, table_sh.at[pl.ds(c * _ZR, _ZR)], sems[i % nsem])
  for i in range(max(0, niter - nsem), niter):
    c = sid + _NS * i

    @pl.when(c < _NZCHUNK)
    def _():
      pltpu.make_async_copy(zsrc, table_sh.at[pl.ds(0, _ZR)],
                            sems[i % nsem]).wait()


def _drain_table(sid, table_sh, out_slice, sems):
  nsem = len(sems)
  niter = pl.cdiv(_NRCHUNK, _NS)
  dummy = pl.ds(0, _RCHUNK)
  for i in range(niter):
    if i >= nsem:
      cprev = sid + _NS * (i - nsem)

      @pl.when(cprev < _NRCHUNK)
      def _():
        pltpu.make_async_copy(table_sh.at[dummy], out_slice.at[dummy],
                              sems[i % nsem]).wait()
    c = sid + _NS * i

    @pl.when(c < _NRCHUNK)
    def _():
      rows_sl = pl.ds(c * _RCHUNK, _RCHUNK)
      pltpu.async_copy(table_sh.at[rows_sl], out_slice.at[rows_sl],
                       sems[i % nsem])
  for i in range(max(0, niter - nsem), niter):
    c = sid + _NS * i

    @pl.when(c < _NRCHUNK)
    def _():
      pltpu.make_async_copy(table_sh.at[dummy], out_slice.at[dummy],
                            sems[i % nsem]).wait()


def _segsum_body(y_hbm, src_hbm, dst_hbm, s_out, *scr):
  cid = lax.axis_index("c")
  sid = lax.axis_index("s")
  wid = cid * _NS + sid
  src_v, dst_v = scr[0], scr[1]
  rows = list(scr[2:2 + _NBUF])
  acc_sh = scr[2 + _NBUF]
  gsem = list(scr[3 + _NBUF:3 + 2 * _NBUF])
  ssem = list(scr[3 + 2 * _NBUF:3 + 3 * _NBUF])

  def _fill_row(i, _):
    for j in range(_D // 16):
      rows[0][i, pl.ds(j * 16, 16)] = jnp.zeros((16,), _f32)
    return 0
  lax.fori_loop(0, _ZR, _fill_row, 0)

  _zero_table(sid, rows[0].at[pl.ds(0, _ZR)], acc_sh, ssem)
  plsc.subcore_barrier()

  def _idx_block(b, _):
    pltpu.async_copy(src_hbm.at[wid, b], src_v, gsem[0])
    pltpu.async_copy(dst_hbm.at[wid, b], dst_v, gsem[1])
    pltpu.make_async_copy(src_hbm.at[wid, b], src_v, gsem[0]).wait()
    pltpu.make_async_copy(dst_hbm.at[wid, b], dst_v, gsem[1]).wait()
    for k in range(_NBUF):
      pltpu.async_copy(y_hbm.at[src_v.at[k]], rows[k], gsem[k])

    def _group(i, _):
      base = i * _NBUF
      for k in range(_NBUF):
        c = base + k
        pltpu.make_async_copy(y_hbm.at[src_v.at[c]], rows[k], gsem[k]).wait()
        pltpu.async_copy(rows[k], acc_sh.at[dst_v.at[c]], ssem[k], add=True)
      for k in range(_NBUF):
        c = base + k + _NBUF

        @pl.when(c < _IBLK)
        def _():
          pltpu.make_async_copy(
              rows[k], acc_sh.at[dst_v.at[c]], ssem[k]).wait()
          pltpu.async_copy(y_hbm.at[src_v.at[c]], rows[k], gsem[k])
      return 0
    lax.fori_loop(0, _IBLK // _NBUF, _group, 0)

    for k in range(_NBUF):
      pltpu.make_async_copy(
          rows[k], acc_sh.at[dst_v.at[0]], ssem[k]).wait()
    return 0
  lax.fori_loop(0, _NBLK, _idx_block, 0)
  plsc.subcore_barrier()

  _drain_table(sid, acc_sh, s_out.at[cid], ssem)


_segsum = pl.kernel(
    _segsum_body,
    out_type=jax.ShapeDtypeStruct((_NC, _N, _D), _f32),
    mesh=plsc.VectorSubcoreMesh(core_axis_name="c", subcore_axis_name="s"),
    scratch_types=(
        [pltpu.VMEM((_IBLK, _CHUNK), jnp.int32)] * 2
        + [pltpu.VMEM((_CHUNK, _D), _f32)] * _NBUF
        + [pltpu.VMEM_SHARED((_N, _D), _f32)]
        + [pltpu.SemaphoreType.DMA] * (2 * _NBUF)
    ),
)


def _count_body(dst_hbm, cnt_out, *scr):
  cid = lax.axis_index("c")
  sid = lax.axis_index("s")
  wid = cid * _NS + sid
  dst_v, ones_v, zcnt_v = scr[0], scr[1], scr[2]
  cnt_sh = scr[3]
  ssem = list(scr[4:4 + _NBUF])

  def _fill_z(i, _):
    for j in range(_CNTW // 16):
      zcnt_v[i, pl.ds(j * 16, 16)] = jnp.zeros((16,), _f32)
    return 0
  lax.fori_loop(0, _ZR, _fill_z, 0)

  def _fill_o(i, _):
    for j in range(_CNTW // 16):
      ones_v[i, pl.ds(j * 16, 16)] = jnp.ones((16,), _f32)
    return 0
  lax.fori_loop(0, _CCHUNK, _fill_o, 0)

  for r in range(4):
    _zero_table(sid, zcnt_v.at[pl.ds(0, _ZR)], cnt_sh, ssem)
    plsc.subcore_barrier()

    def _idx_block(b, _):
      pltpu.sync_copy(dst_hbm.at[r, wid, b], dst_v)
      for k in range(_NBUF):
        pltpu.async_copy(ones_v, cnt_sh.at[dst_v.at[k]], ssem[k], add=True)

      def _group(i, _):
        base = i * _NBUF
        for k in range(_NBUF):
          c = base + k + _NBUF

          @pl.when(c < _CIBLK)
          def _():
            pltpu.make_async_copy(
                ones_v, cnt_sh.at[dst_v.at[c]], ssem[k]).wait()
            pltpu.async_copy(ones_v, cnt_sh.at[dst_v.at[c]], ssem[k], add=True)
        return 0
      lax.fori_loop(0, _CIBLK // _NBUF, _group, 0)

      for k in range(_NBUF):
        pltpu.make_async_copy(ones_v, cnt_sh.at[dst_v.at[0]], ssem[k]).wait()
      return 0
    lax.fori_loop(0, _CNBLK, _idx_block, 0)
    plsc.subcore_barrier()

    _drain_table(sid, cnt_sh, cnt_out.at[r, cid], ssem)
    plsc.subcore_barrier()


_count = pl.kernel(
    _count_body,
    out_type=jax.ShapeDtypeStruct((4, _NC, _N, _CNTW), _f32),
    mesh=plsc.VectorSubcoreMesh(core_axis_name="c", subcore_axis_name="s"),
    scratch_types=(
        [pltpu.VMEM((_CIBLK, _CCHUNK), jnp.int32)]
        + [pltpu.VMEM((_CCHUNK, _CNTW), _f32)]
        + [pltpu.VMEM((_ZR, _CNTW), _f32)]
        + [pltpu.VMEM_SHARED((_N, _CNTW), _f32)]
        + [pltpu.SemaphoreType.DMA] * _NBUF
    ),
)


def _mm_body(x_ref, w_ref, o_ref):
  o_ref[...] = jnp.dot(x_ref[...], w_ref[...], preferred_element_type=_f32)


_mm = pl.pallas_call(
    _mm_body,
    out_shape=jax.ShapeDtypeStruct((_N, _D), _f32),
)


def _leaky(v):
  return jnp.where(v >= 0, v, 0.01 * v)


def _comb1_body(s_ref, c_ref, x_ref, w_ref, b_ref, o_ref):
  stot = s_ref[0] + s_ref[1]
  inv = 1.0 / jnp.maximum(c_ref[0] + c_ref[1], 1.0)
  v = (stot * inv + b_ref[...]
       + jnp.dot(x_ref[...], w_ref[...], preferred_element_type=_f32))
  o_ref[...] = _leaky(v)


_comb1 = pl.pallas_call(
    _comb1_body,
    out_shape=jax.ShapeDtypeStruct((_N, _D), _f32),
)


def _comb2_body(sa_ref, ca_ref, sb_ref, cb_ref, x_ref, w_ref, b_ref, o_ref):
  sa = sa_ref[0] + sa_ref[1]
  ca = ca_ref[0] + ca_ref[1]
  sb = sb_ref[0] + sb_ref[1]
  cb = cb_ref[0] + cb_ref[1]
  v = (sa / jnp.maximum(ca, 1.0) + sb / jnp.maximum(cb, 1.0) + b_ref[...]
       + jnp.dot(x_ref[...], w_ref[...], preferred_element_type=_f32))
  o_ref[...] = _leaky(v)


_comb2 = pl.pallas_call(
    _comb2_body,
    out_shape=jax.ShapeDtypeStruct((_N, _D), _f32),
)


def kernel(x_node1, x_node2, x_node3, edge_index_node2_to_node3,
           edge_index_node1_to_node2, edge_index_node3_rev_to_node2,
           edge_index_node2_rev_to_node1, params):
  edges = [edge_index_node2_to_node3, edge_index_node1_to_node2,
           edge_index_node3_rev_to_node2, edge_index_node2_rev_to_node1]
  srcs, dsts, dsts_c = [], [], []
  for e in edges:
    e = e.astype(jnp.int32)
    srcs.append(e[0].reshape(_NW, _NBLK, _IBLK, _CHUNK))
    dsts.append(e[1].reshape(_NW, _NBLK, _IBLK, _CHUNK))
    dsts_c.append(e[1].reshape(_NW, _CNBLK, _CIBLK, _CCHUNK))

  cnt = _count(jnp.stack(dsts_c))

  rel = [(1, 2), (0, 1), (2, 1), (1, 0)]
  xs = (x_node1, x_node2, x_node3)

  for li in range(2):
    lp = params["layer%d" % li]
    segs = []
    for r in range(4):
      p = lp["e%d" % r]
      y = _mm(xs[rel[r][0]], p["W_l"])
      segs.append(_segsum(y, srcs[r], dsts[r]))

    b = [lp["e%d" % r]["b_l"].reshape(1, _D) for r in range(4)]
    new1 = _comb1(segs[3], cnt[3], xs[0], lp["e3"]["W_r"], b[3])
    new2 = _comb2(segs[1], cnt[1], segs[2], cnt[2], xs[1],
                  lp["e1"]["W_r"] + lp["e2"]["W_r"], b[1] + b[2])
    new3 = _comb1(segs[0], cnt[0], xs[2], lp["e0"]["W_r"], b[0])
    xs = (new1, new2, new3)

  return xs


# speedup vs baseline: 1.1841x; 1.0003x over previous
import jax
import jax.numpy as jnp
from jax import lax
from jax.experimental import pallas as pl
from jax.experimental.pallas import tpu as pltpu
from jax.experimental.pallas import tpu_sc as plsc

_NC = 2
_NS = 16
_NW = _NC * _NS

_N = 10000
_D = 128
_E = 320000

_CHUNK = 50
_EPT = _E // _NW
_NCHUNK = _EPT // _CHUNK
_NBUF = 5
_RCHUNK = 80
_NRCHUNK = _N // _RCHUNK
_CNTW = 128
_IBLK = 50
_NBLK = _NCHUNK // _IBLK
_ZR = 40
_NZCHUNK = _N // _ZR

_CCHUNK = 125
_CIBLK = 16
_CNBLK = (_EPT // _CCHUNK) // _CIBLK

_f32 = jnp.float32


def _zero_table(sid, zsrc, table_sh, sems, zr=_ZR):
  nsem = len(sems)
  nzchunk = _N // zr
  niter = pl.cdiv(nzchunk, _NS)
  for i in range(niter):
    if i >= nsem:
      cprev = sid + _NS * (i - nsem)

      @pl.when(cprev < nzchunk)
      def _():
        pltpu.make_async_copy(zsrc, table_sh.at[pl.ds(0, zr)],
                              sems[i % nsem]).wait()
    c = sid + _NS * i

    @pl.when(c < nzchunk)
    def _():
      pltpu.async_copy(zsrc, table_sh.at[pl.ds(c * zr, zr)], sems[i % nsem])
  for i in range(max(0, niter - nsem), niter):
    c = sid + _NS * i

    @pl.when(c < nzchunk)
    def _():
      pltpu.make_async_copy(zsrc, table_sh.at[pl.ds(0, zr)],
                            sems[i % nsem]).wait()


def _drain_table(sid, table_sh, out_slice, sems):
  nsem = len(sems)
  niter = pl.cdiv(_NRCHUNK, _NS)
  dummy = pl.ds(0, _RCHUNK)
  for i in range(niter):
    if i >= nsem:
      cprev = sid + _NS * (i - nsem)

      @pl.when(cprev < _NRCHUNK)
      def _():
        pltpu.make_async_copy(table_sh.at[dummy], out_slice.at[dummy],
                              sems[i % nsem]).wait()
    c = sid + _NS * i

    @pl.when(c < _NRCHUNK)
    def _():
      rows_sl = pl.ds(c * _RCHUNK, _RCHUNK)
      pltpu.async_copy(table_sh.at[rows_sl], out_slice.at[rows_sl],
                       sems[i % nsem])
  for i in range(max(0, niter - nsem), niter):
    c = sid + _NS * i

    @pl.when(c < _NRCHUNK)
    def _():
      pltpu.make_async_copy(table_sh.at[dummy], out_slice.at[dummy],
                            sems[i % nsem]).wait()


def _segsum_body(y_hbm, src_hbm, dst_hbm, s_out, *scr):
  cid = lax.axis_index("c")
  sid = lax.axis_index("s")
  wid = cid * _NS + sid
  src_v, dst_v = scr[0], scr[1]
  rows = list(scr[2:2 + _NBUF])
  acc_sh = scr[2 + _NBUF]
  gsem = list(scr[3 + _NBUF:3 + 2 * _NBUF])
  ssem = list(scr[3 + 2 * _NBUF:3 + 3 * _NBUF])

  def _fill_row(i, _):
    for j in range(_D // 16):
      rows[0][i, pl.ds(j * 16, 16)] = jnp.zeros((16,), _f32)
    return 0
  lax.fori_loop(0, _ZR, _fill_row, 0)

  _zero_table(sid, rows[0].at[pl.ds(0, _ZR)], acc_sh, ssem)
  plsc.subcore_barrier()

  def _idx_block(b, _):
    pltpu.async_copy(src_hbm.at[wid, b], src_v, gsem[0])
    pltpu.async_copy(dst_hbm.at[wid, b], dst_v, gsem[1])
    pltpu.make_async_copy(src_hbm.at[wid, b], src_v, gsem[0]).wait()
    pltpu.make_async_copy(dst_hbm.at[wid, b], dst_v, gsem[1]).wait()
    for k in range(_NBUF):
      pltpu.async_copy(y_hbm.at[src_v.at[k]], rows[k], gsem[k])

    def _group(i, _):
      base = i * _NBUF
      for k in range(_NBUF):
        c = base + k
        pltpu.make_async_copy(y_hbm.at[src_v.at[c]], rows[k], gsem[k]).wait()
        pltpu.async_copy(rows[k], acc_sh.at[dst_v.at[c]], ssem[k], add=True)
      for k in range(_NBUF):
        c = base + k + _NBUF

        @pl.when(c < _IBLK)
        def _():
          pltpu.make_async_copy(
              rows[k], acc_sh.at[dst_v.at[c]], ssem[k]).wait()
          pltpu.async_copy(y_hbm.at[src_v.at[c]], rows[k], gsem[k])
      return 0
    lax.fori_loop(0, _IBLK // _NBUF, _group, 0)

    for k in range(_NBUF):
      pltpu.make_async_copy(
          rows[k], acc_sh.at[dst_v.at[0]], ssem[k]).wait()
    return 0
  lax.fori_loop(0, _NBLK, _idx_block, 0)
  plsc.subcore_barrier()

  _drain_table(sid, acc_sh, s_out.at[cid], ssem)


_segsum = pl.kernel(
    _segsum_body,
    out_type=jax.ShapeDtypeStruct((_NC, _N, _D), _f32),
    mesh=plsc.VectorSubcoreMesh(core_axis_name="c", subcore_axis_name="s"),
    scratch_types=(
        [pltpu.VMEM((_IBLK, _CHUNK), jnp.int32)] * 2
        + [pltpu.VMEM((_CHUNK, _D), _f32)] * _NBUF
        + [pltpu.VMEM_SHARED((_N, _D), _f32)]
        + [pltpu.SemaphoreType.DMA] * (2 * _NBUF)
    ),
)


def _count_body(dst_hbm, cnt_out, *scr):
  cid = lax.axis_index("c")
  sid = lax.axis_index("s")
  wid = cid * _NS + sid
  dst_v, ones_v, zcnt_v = scr[0], scr[1], scr[2]
  cnt_sh = scr[3]
  ssem = list(scr[4:4 + _NBUF])

  def _fill_z(i, _):
    for j in range(_CNTW // 16):
      zcnt_v[i, pl.ds(j * 16, 16)] = jnp.zeros((16,), _f32)
    return 0
  lax.fori_loop(0, _ZR, _fill_z, 0)

  def _fill_o(i, _):
    for j in range(_CNTW // 16):
      ones_v[i, pl.ds(j * 16, 16)] = jnp.ones((16,), _f32)
    return 0
  lax.fori_loop(0, _CCHUNK, _fill_o, 0)

  for r in range(4):
    _zero_table(sid, zcnt_v.at[pl.ds(0, _ZR)], cnt_sh, ssem)
    plsc.subcore_barrier()

    def _idx_block(b, _):
      pltpu.sync_copy(dst_hbm.at[r, wid, b], dst_v)
      for k in range(_NBUF):
        pltpu.async_copy(ones_v, cnt_sh.at[dst_v.at[k]], ssem[k], add=True)

      def _group(i, _):
        base = i * _NBUF
        for k in range(_NBUF):
          c = base + k + _NBUF

          @pl.when(c < _CIBLK)
          def _():
            pltpu.make_async_copy(
                ones_v, cnt_sh.at[dst_v.at[c]], ssem[k]).wait()
            pltpu.async_copy(ones_v, cnt_sh.at[dst_v.at[c]], ssem[k], add=True)
        return 0
      lax.fori_loop(0, _CIBLK // _NBUF, _group, 0)

      for k in range(_NBUF):
        pltpu.make_async_copy(ones_v, cnt_sh.at[dst_v.at[0]], ssem[k]).wait()
      return 0
    lax.fori_loop(0, _CNBLK, _idx_block, 0)
    plsc.subcore_barrier()

    _drain_table(sid, cnt_sh, cnt_out.at[r, cid], ssem)
    plsc.subcore_barrier()


_count = pl.kernel(
    _count_body,
    out_type=jax.ShapeDtypeStruct((4, _NC, _N, _CNTW), _f32),
    mesh=plsc.VectorSubcoreMesh(core_axis_name="c", subcore_axis_name="s"),
    scratch_types=(
        [pltpu.VMEM((_CIBLK, _CCHUNK), jnp.int32)]
        + [pltpu.VMEM((_CCHUNK, _CNTW), _f32)]
        + [pltpu.VMEM((_ZR, _CNTW), _f32)]
        + [pltpu.VMEM_SHARED((_N, _CNTW), _f32)]
        + [pltpu.SemaphoreType.DMA] * _NBUF
    ),
)


def _mm_body(x_ref, w_ref, o_ref):
  o_ref[...] = jnp.dot(x_ref[...], w_ref[...], preferred_element_type=_f32)


_mm = pl.pallas_call(
    _mm_body,
    out_shape=jax.ShapeDtypeStruct((_N, _D), _f32),
)


def _leaky(v):
  return jnp.where(v >= 0, v, 0.01 * v)


def _comb1_body(s_ref, c_ref, x_ref, w_ref, b_ref, o_ref):
  stot = s_ref[0] + s_ref[1]
  inv = 1.0 / jnp.maximum(c_ref[0] + c_ref[1], 1.0)
  v = (stot * inv + b_ref[...]
       + jnp.dot(x_ref[...], w_ref[...], preferred_element_type=_f32))
  o_ref[...] = _leaky(v)


_comb1 = pl.pallas_call(
    _comb1_body,
    out_shape=jax.ShapeDtypeStruct((_N, _D), _f32),
)


def _comb2_body(sa_ref, ca_ref, sb_ref, cb_ref, x_ref, w_ref, b_ref, o_ref):
  sa = sa_ref[0] + sa_ref[1]
  ca = ca_ref[0] + ca_ref[1]
  sb = sb_ref[0] + sb_ref[1]
  cb = cb_ref[0] + cb_ref[1]
  v = (sa / jnp.maximum(ca, 1.0) + sb / jnp.maximum(cb, 1.0) + b_ref[...]
       + jnp.dot(x_ref[...], w_ref[...], preferred_element_type=_f32))
  o_ref[...] = _leaky(v)


_comb2 = pl.pallas_call(
    _comb2_body,
    out_shape=jax.ShapeDtypeStruct((_N, _D), _f32),
)


def kernel(x_node1, x_node2, x_node3, edge_index_node2_to_node3,
           edge_index_node1_to_node2, edge_index_node3_rev_to_node2,
           edge_index_node2_rev_to_node1, params):
  edges = [edge_index_node2_to_node3, edge_index_node1_to_node2,
           edge_index_node3_rev_to_node2, edge_index_node2_rev_to_node1]
  srcs, dsts, dsts_c = [], [], []
  for e in edges:
    e = e.astype(jnp.int32)
    srcs.append(e[0].reshape(_NW, _NBLK, _IBLK, _CHUNK))
    dsts.append(e[1].reshape(_NW, _NBLK, _IBLK, _CHUNK))
    dsts_c.append(e[1].reshape(_NW, _CNBLK, _CIBLK, _CCHUNK))

  cnt = _count(jnp.stack(dsts_c))

  rel = [(1, 2), (0, 1), (2, 1), (1, 0)]
  xs = (x_node1, x_node2, x_node3)

  for li in range(2):
    lp = params["layer%d" % li]
    segs = []
    for r in range(4):
      p = lp["e%d" % r]
      y = _mm(xs[rel[r][0]], p["W_l"])
      segs.append(_segsum(y, srcs[r], dsts[r]))

    b = [lp["e%d" % r]["b_l"].reshape(1, _D) for r in range(4)]
    new1 = _comb1(segs[3], cnt[3], xs[0], lp["e3"]["W_r"], b[3])
    new2 = _comb2(segs[1], cnt[1], segs[2], cnt[2], xs[1],
                  lp["e1"]["W_r"] + lp["e2"]["W_r"], b[1] + b[2])
    new3 = _comb1(segs[0], cnt[0], xs[2], lp["e0"]["W_r"], b[0])
    xs = (new1, new2, new3)

  return xs


# speedup vs baseline: 1.1951x; 1.0093x over previous
import jax
import jax.numpy as jnp
from jax import lax
from jax.experimental import pallas as pl
from jax.experimental.pallas import tpu as pltpu
from jax.experimental.pallas import tpu_sc as plsc

_NC = 2
_NS = 16
_NW = _NC * _NS

_N = 10000
_D = 128
_E = 320000

_CHUNK = 50
_EPT = _E // _NW
_NCHUNK = _EPT // _CHUNK
_NBUF = 5
_RCHUNK = 80
_NRCHUNK = _N // _RCHUNK
_CNTW = 128
_IBLK = 50
_NBLK = _NCHUNK // _IBLK
_ZR = 40
_NZCHUNK = _N // _ZR

_CCHUNK = 125
_CIBLK = 80
_CNBLK = (_EPT // _CCHUNK) // _CIBLK

_f32 = jnp.float32


def _zero_table(sid, zsrc, table_sh, sems, zr=_ZR):
  nsem = len(sems)
  nzchunk = _N // zr
  niter = pl.cdiv(nzchunk, _NS)
  for i in range(niter):
    if i >= nsem:
      cprev = sid + _NS * (i - nsem)

      @pl.when(cprev < nzchunk)
      def _():
        pltpu.make_async_copy(zsrc, table_sh.at[pl.ds(0, zr)],
                              sems[i % nsem]).wait()
    c = sid + _NS * i

    @pl.when(c < nzchunk)
    def _():
      pltpu.async_copy(zsrc, table_sh.at[pl.ds(c * zr, zr)], sems[i % nsem])
  for i in range(max(0, niter - nsem), niter):
    c = sid + _NS * i

    @pl.when(c < nzchunk)
    def _():
      pltpu.make_async_copy(zsrc, table_sh.at[pl.ds(0, zr)],
                            sems[i % nsem]).wait()


def _drain_table(sid, table_sh, out_slice, sems):
  nsem = len(sems)
  niter = pl.cdiv(_NRCHUNK, _NS)
  dummy = pl.ds(0, _RCHUNK)
  for i in range(niter):
    if i >= nsem:
      cprev = sid + _NS * (i - nsem)

      @pl.when(cprev < _NRCHUNK)
      def _():
        pltpu.make_async_copy(table_sh.at[dummy], out_slice.at[dummy],
                              sems[i % nsem]).wait()
    c = sid + _NS * i

    @pl.when(c < _NRCHUNK)
    def _():
      rows_sl = pl.ds(c * _RCHUNK, _RCHUNK)
      pltpu.async_copy(table_sh.at[rows_sl], out_slice.at[rows_sl],
                       sems[i % nsem])
  for i in range(max(0, niter - nsem), niter):
    c = sid + _NS * i

    @pl.when(c < _NRCHUNK)
    def _():
      pltpu.make_async_copy(table_sh.at[dummy], out_slice.at[dummy],
                            sems[i % nsem]).wait()


def _segsum_body(y_hbm, src_hbm, dst_hbm, s_out, *scr):
  cid = lax.axis_index("c")
  sid = lax.axis_index("s")
  wid = cid * _NS + sid
  src_v, dst_v = scr[0], scr[1]
  rows = list(scr[2:2 + _NBUF])
  acc_sh = scr[2 + _NBUF]
  gsem = list(scr[3 + _NBUF:3 + 2 * _NBUF])
  ssem = list(scr[3 + 2 * _NBUF:3 + 3 * _NBUF])

  def _fill_row(i, _):
    for j in range(_D // 16):
      rows[0][i, pl.ds(j * 16, 16)] = jnp.zeros((16,), _f32)
    return 0
  lax.fori_loop(0, _ZR, _fill_row, 0)

  _zero_table(sid, rows[0].at[pl.ds(0, _ZR)], acc_sh, ssem)
  plsc.subcore_barrier()

  def _idx_block(b, _):
    pltpu.async_copy(src_hbm.at[wid, b], src_v, gsem[0])
    pltpu.async_copy(dst_hbm.at[wid, b], dst_v, gsem[1])
    pltpu.make_async_copy(src_hbm.at[wid, b], src_v, gsem[0]).wait()
    pltpu.make_async_copy(dst_hbm.at[wid, b], dst_v, gsem[1]).wait()
    for k in range(_NBUF):
      pltpu.async_copy(y_hbm.at[src_v.at[k]], rows[k], gsem[k])

    def _group(i, _):
      base = i * _NBUF
      for k in range(_NBUF):
        c = base + k
        pltpu.make_async_copy(y_hbm.at[src_v.at[c]], rows[k], gsem[k]).wait()
        pltpu.async_copy(rows[k], acc_sh.at[dst_v.at[c]], ssem[k], add=True)
      for k in range(_NBUF):
        c = base + k + _NBUF

        @pl.when(c < _IBLK)
        def _():
          pltpu.make_async_copy(
              rows[k], acc_sh.at[dst_v.at[c]], ssem[k]).wait()
          pltpu.async_copy(y_hbm.at[src_v.at[c]], rows[k], gsem[k])
      return 0
    lax.fori_loop(0, _IBLK // _NBUF, _group, 0)

    for k in range(_NBUF):
      pltpu.make_async_copy(
          rows[k], acc_sh.at[dst_v.at[0]], ssem[k]).wait()
    return 0
  lax.fori_loop(0, _NBLK, _idx_block, 0)
  plsc.subcore_barrier()

  _drain_table(sid, acc_sh, s_out.at[cid], ssem)


_segsum = pl.kernel(
    _segsum_body,
    out_type=jax.ShapeDtypeStruct((_NC, _N, _D), _f32),
    mesh=plsc.VectorSubcoreMesh(core_axis_name="c", subcore_axis_name="s"),
    scratch_types=(
        [pltpu.VMEM((_IBLK, _CHUNK), jnp.int32)] * 2
        + [pltpu.VMEM((_CHUNK, _D), _f32)] * _NBUF
        + [pltpu.VMEM_SHARED((_N, _D), _f32)]
        + [pltpu.SemaphoreType.DMA] * (2 * _NBUF)
    ),
)


def _count_body(dst_hbm, cnt_out, *scr):
  cid = lax.axis_index("c")
  sid = lax.axis_index("s")
  wid = cid * _NS + sid
  dst_v, ones_v, zcnt_v = scr[0], scr[1], scr[2]
  cnt_sh = scr[3]
  ssem = list(scr[4:4 + _NBUF])

  def _fill_z(i, _):
    for j in range(_CNTW // 16):
      zcnt_v[i, pl.ds(j * 16, 16)] = jnp.zeros((16,), _f32)
    return 0
  lax.fori_loop(0, _ZR, _fill_z, 0)

  def _fill_o(i, _):
    for j in range(_CNTW // 16):
      ones_v[i, pl.ds(j * 16, 16)] = jnp.ones((16,), _f32)
    return 0
  lax.fori_loop(0, _CCHUNK, _fill_o, 0)

  for r in range(4):
    _zero_table(sid, zcnt_v.at[pl.ds(0, _ZR)], cnt_sh, ssem)
    plsc.subcore_barrier()

    def _idx_block(b, _):
      pltpu.sync_copy(dst_hbm.at[r, wid, b], dst_v)
      for k in range(_NBUF):
        pltpu.async_copy(ones_v, cnt_sh.at[dst_v.at[k]], ssem[k], add=True)

      def _group(i, _):
        base = i * _NBUF
        for k in range(_NBUF):
          c = base + k + _NBUF

          @pl.when(c < _CIBLK)
          def _():
            pltpu.make_async_copy(
                ones_v, cnt_sh.at[dst_v.at[c]], ssem[k]).wait()
            pltpu.async_copy(ones_v, cnt_sh.at[dst_v.at[c]], ssem[k], add=True)
        return 0
      lax.fori_loop(0, _CIBLK // _NBUF, _group, 0)

      for k in range(_NBUF):
        pltpu.make_async_copy(ones_v, cnt_sh.at[dst_v.at[0]], ssem[k]).wait()
      return 0
    lax.fori_loop(0, _CNBLK, _idx_block, 0)
    plsc.subcore_barrier()

    _drain_table(sid, cnt_sh, cnt_out.at[r, cid], ssem)
    plsc.subcore_barrier()


_count = pl.kernel(
    _count_body,
    out_type=jax.ShapeDtypeStruct((4, _NC, _N, _CNTW), _f32),
    mesh=plsc.VectorSubcoreMesh(core_axis_name="c", subcore_axis_name="s"),
    scratch_types=(
        [pltpu.VMEM((_CIBLK, _CCHUNK), jnp.int32)]
        + [pltpu.VMEM((_CCHUNK, _CNTW), _f32)]
        + [pltpu.VMEM((_ZR, _CNTW), _f32)]
        + [pltpu.VMEM_SHARED((_N, _CNTW), _f32)]
        + [pltpu.SemaphoreType.DMA] * _NBUF
    ),
)


def _mm_body(x_ref, w_ref, o_ref):
  o_ref[...] = jnp.dot(x_ref[...], w_ref[...], preferred_element_type=_f32)


_mm = pl.pallas_call(
    _mm_body,
    out_shape=jax.ShapeDtypeStruct((_N, _D), _f32),
)


def _leaky(v):
  return jnp.where(v >= 0, v, 0.01 * v)


def _comb1_body(s_ref, c_ref, x_ref, w_ref, b_ref, o_ref):
  stot = s_ref[0] + s_ref[1]
  inv = 1.0 / jnp.maximum(c_ref[0] + c_ref[1], 1.0)
  v = (stot * inv + b_ref[...]
       + jnp.dot(x_ref[...], w_ref[...], preferred_element_type=_f32))
  o_ref[...] = _leaky(v)


_comb1 = pl.pallas_call(
    _comb1_body,
    out_shape=jax.ShapeDtypeStruct((_N, _D), _f32),
)


def _comb2_body(sa_ref, ca_ref, sb_ref, cb_ref, x_ref, w_ref, b_ref, o_ref):
  sa = sa_ref[0] + sa_ref[1]
  ca = ca_ref[0] + ca_ref[1]
  sb = sb_ref[0] + sb_ref[1]
  cb = cb_ref[0] + cb_ref[1]
  v = (sa / jnp.maximum(ca, 1.0) + sb / jnp.maximum(cb, 1.0) + b_ref[...]
       + jnp.dot(x_ref[...], w_ref[...], preferred_element_type=_f32))
  o_ref[...] = _leaky(v)


_comb2 = pl.pallas_call(
    _comb2_body,
    out_shape=jax.ShapeDtypeStruct((_N, _D), _f32),
)


def kernel(x_node1, x_node2, x_node3, edge_index_node2_to_node3,
           edge_index_node1_to_node2, edge_index_node3_rev_to_node2,
           edge_index_node2_rev_to_node1, params):
  edges = [edge_index_node2_to_node3, edge_index_node1_to_node2,
           edge_index_node3_rev_to_node2, edge_index_node2_rev_to_node1]
  srcs, dsts, dsts_c = [], [], []
  for e in edges:
    e = e.astype(jnp.int32)
    srcs.append(e[0].reshape(_NW, _NBLK, _IBLK, _CHUNK))
    dsts.append(e[1].reshape(_NW, _NBLK, _IBLK, _CHUNK))
    dsts_c.append(e[1].reshape(_NW, _CNBLK, _CIBLK, _CCHUNK))

  cnt = _count(jnp.stack(dsts_c))

  rel = [(1, 2), (0, 1), (2, 1), (1, 0)]
  xs = (x_node1, x_node2, x_node3)

  for li in range(2):
    lp = params["layer%d" % li]
    segs = []
    for r in range(4):
      p = lp["e%d" % r]
      y = _mm(xs[rel[r][0]], p["W_l"])
      segs.append(_segsum(y, srcs[r], dsts[r]))

    b = [lp["e%d" % r]["b_l"].reshape(1, _D) for r in range(4)]
    new1 = _comb1(segs[3], cnt[3], xs[0], lp["e3"]["W_r"], b[3])
    new2 = _comb2(segs[1], cnt[1], segs[2], cnt[2], xs[1],
                  lp["e1"]["W_r"] + lp["e2"]["W_r"], b[1] + b[2])
    new3 = _comb1(segs[0], cnt[0], xs[2], lp["e0"]["W_r"], b[0])
    xs = (new1, new2, new3)

  return xs
